# baseline (speedup 1.0000x reference)
import functools
import math

import jax
import jax.numpy as jnp
from jax import lax
from jax.experimental import pallas as pl
from jax.experimental.pallas import tpu as pltpu
from jax.experimental.pallas import tpu_sc as plsc

_N = 10000
_E = 320000
_D = 128
_H = 128
_C = 64
_ALPHA = 0.1
_LAMBDA = 0.5

_NC = 2
_NS = 16
_NW = _NC * _NS

_CH = 128
_CPT = 80
_EPAD = _NW * _CPT * _CH
_NPAD = 10240
_ZRD = _NPAD // _NS

_CHA = 128
_CPA = 80
_EPA = _NW * _CPA * _CHA
_MR = 10016
_MZR = _MR // _NS
_ZB = 16
_PK = 14


def _sc_mesh():
    return plsc.VectorSubcoreMesh(
        core_axis_name="c", subcore_axis_name="s",
        num_cores=_NC, num_subcores=_NS)


@functools.partial(
    pl.kernel,
    out_type=jax.ShapeDtypeStruct((_NC, 2, _NPAD), jnp.float32),
    mesh=_sc_mesh(),
    scratch_types=[
        pltpu.VMEM((_CPT, _CH), jnp.int32),
        pltpu.VMEM((_CPT, _CH), jnp.int32),
        pltpu.VMEM((_CH,), jnp.float32),
        pltpu.VMEM((_ZRD,), jnp.float32),
        pltpu.VMEM_SHARED((_NPAD,), jnp.float32),
        pltpu.VMEM_SHARED((_NPAD,), jnp.float32),
    ],
)
def _deg_kernel(srcs, dsts, out, src_v, dst_v, ones_v, zer_v, do_sh, di_sh):
    cid = lax.axis_index("c")
    sid = lax.axis_index("s")
    wid = cid * _NS + sid
    for k in range(_CH // 16):
        ones_v[pl.ds(k * 16, 16)] = jnp.full((16,), 1.0, jnp.float32)
    for k in range(_ZRD // 16):
        zer_v[pl.ds(k * 16, 16)] = jnp.zeros((16,), jnp.float32)
    pltpu.sync_copy(zer_v, do_sh.at[pl.ds(sid * _ZRD, _ZRD)])
    pltpu.sync_copy(zer_v, di_sh.at[pl.ds(sid * _ZRD, _ZRD)])
    pltpu.sync_copy(srcs.at[wid], src_v)
    pltpu.sync_copy(dsts.at[wid], dst_v)
    plsc.subcore_barrier()

    def body(j, carry):
        pltpu.sync_copy(ones_v, do_sh.at[src_v.at[j]], add=True)
        pltpu.sync_copy(ones_v, di_sh.at[dst_v.at[j]], add=True)
        return carry

    lax.fori_loop(0, _CPT, body, 0)
    plsc.subcore_barrier()
    rows = pl.ds(sid * _ZRD, _ZRD)
    pltpu.sync_copy(do_sh.at[rows], out.at[cid, 0, rows])
    pltpu.sync_copy(di_sh.at[rows], out.at[cid, 1, rows])


@functools.partial(
    pl.kernel,
    out_type=jax.ShapeDtypeStruct((_NC, _MR, _H), jnp.float32),
    mesh=_sc_mesh(),
    scratch_types=[
        pltpu.VMEM((_CPA, _CHA), jnp.int32),
        pltpu.VMEM((2, _CHA), jnp.int32),
        pltpu.VMEM((2, _CHA), jnp.int32),
        pltpu.VMEM((_ZB, _H), jnp.float32),
        pltpu.VMEM((_CHA, _H), jnp.float32),
        pltpu.VMEM((_CHA, _H), jnp.float32),
        pltpu.VMEM_SHARED((_MR, _H), jnp.float32),
        pltpu.SemaphoreType.DMA,
        pltpu.SemaphoreType.DMA,
        pltpu.SemaphoreType.DMA,
        pltpu.SemaphoreType.DMA,
    ],
    compiler_params=pltpu.CompilerParams(use_tc_tiling_on_sc=False),
)
def _agg_kernel(m_hbm, epk, out_m,
                pk_v, sring, dring, zbuf, rows0, rows1,
                acc_sh, sg0, sg1, sl0, sl1):
    rows = (rows0, rows1)
    sg = (sg0, sg1)
    sl = (sl0, sl1)
    cid = lax.axis_index("c")
    sid = lax.axis_index("s")
    wid = cid * _NS + sid
    pltpu.sync_copy(epk.at[wid], pk_v)

    def zb_body(r, carry):
        for k in range(_H // 16):
            zbuf[r, pl.ds(k * 16, 16)] = jnp.zeros((16,), jnp.float32)
        return carry

    lax.fori_loop(0, _ZB, zb_body, 0)
    base = sid * _MZR
    nfull = _MZR // _ZB
    for r in range(nfull):
        pltpu.sync_copy(zbuf, acc_sh.at[pl.ds(base + r * _ZB, _ZB)])
    rem = _MZR - nfull * _ZB
    if rem:
        pltpu.sync_copy(zbuf.at[pl.ds(0, rem)],
                        acc_sh.at[pl.ds(base + nfull * _ZB, rem)])
    plsc.subcore_barrier()

    def _unpack(j, b):
        for k in range(_CHA // 16):
            v = pk_v[j, pl.ds(k * 16, 16)]
            sring[b, pl.ds(k * 16, 16)] = jnp.bitwise_and(v, (1 << _PK) - 1)
            dring[b, pl.ds(k * 16, 16)] = jnp.right_shift(v, _PK)

    def _start_gather(j, b):
        _unpack(j, b)
        pltpu.async_copy(m_hbm.at[sring.at[b]], rows[b], sg[b])

    def _wait_gather(b):
        pltpu.make_async_copy(m_hbm.at[sring.at[b]], rows[b], sg[b]).wait()

    def _wait_scatter(b):
        pltpu.make_async_copy(rows[b], acc_sh.at[dring.at[b]], sl[b]).wait()

    def _start_scatter(b):
        pltpu.async_copy(rows[b], acc_sh.at[dring.at[b]], sl[b], add=True)

    _start_gather(0, 0)

    def body(g, carry):
        _wait_gather(0)

        @pl.when(g > 0)
        def _():
            _wait_scatter(1)

        _start_gather(g * 2 + 1, 1)
        _start_scatter(0)
        _wait_gather(1)

        @pl.when(g < _CPA // 2 - 1)
        def _():
            _wait_scatter(0)
            _start_gather(g * 2 + 2, 0)

        _start_scatter(1)
        return carry

    lax.fori_loop(0, _CPA // 2, body, 0)
    _wait_scatter(0)
    _wait_scatter(1)
    plsc.subcore_barrier()
    r = pl.ds(sid * _MZR, _MZR)
    pltpu.sync_copy(acc_sh.at[r], out_m.at[cid, r])


_BR = 1000
_GRID = _N // _BR


def _norms(deg_blk):
    deg_out = deg_blk[:, 0] + deg_blk[:, 2]
    deg_in = deg_blk[:, 1] + deg_blk[:, 3]
    ns = lax.rsqrt(jnp.maximum(deg_out, 1.0))
    nd = lax.rsqrt(jnp.maximum(deg_in, 1.0))
    return ns, nd


def _tc_in_body(x_ref, w_ref, b_ref, deg_ref, x0_ref, m_ref):
    h = jnp.dot(x_ref[...], w_ref[...], preferred_element_type=jnp.float32)
    h = jnp.maximum(h + b_ref[...], 0.0)
    ns, _ = _norms(deg_ref[...])
    x0_ref[...] = h
    m_ref[...] = h * ns[:, None]


def _tc_in(x, W0, b0r, degp):
    return pl.pallas_call(
        _tc_in_body,
        grid=(_GRID,),
        in_specs=[
            pl.BlockSpec((_BR, _D), lambda i: (i, 0)),
            pl.BlockSpec((_D, _H), lambda i: (0, 0)),
            pl.BlockSpec((1, _H), lambda i: (0, 0)),
            pl.BlockSpec((_BR, 2 * _NC), lambda i: (i, 0)),
        ],
        out_specs=[
            pl.BlockSpec((_BR, _H), lambda i: (i, 0)),
            pl.BlockSpec((_BR, _H), lambda i: (i, 0)),
        ],
        out_shape=[
            jax.ShapeDtypeStruct((_N, _H), jnp.float32),
            jax.ShapeDtypeStruct((_N, _H), jnp.float32),
        ],
    )(x, W0, b0r, degp)


def _tc_layer_body(beta, p_ref, x0_ref, deg_ref, wc_ref, m_ref):
    ns, nd = _norms(deg_ref[...])
    p = p_ref[...]
    agg = (p[0] + p[1]) * nd[:, None]
    rst = (1.0 - _ALPHA) * agg + _ALPHA * x0_ref[...]
    h = (1.0 - beta) * rst + beta * jnp.dot(
        rst, wc_ref[...], preferred_element_type=jnp.float32)
    h = jnp.maximum(h, 0.0)
    m_ref[...] = h * ns[:, None]


def _tc_layer(beta, p, x0, degp, Wc):
    return pl.pallas_call(
        functools.partial(_tc_layer_body, beta),
        grid=(_GRID,),
        in_specs=[
            pl.BlockSpec((_NC, _BR, _H), lambda i: (0, i, 0)),
            pl.BlockSpec((_BR, _H), lambda i: (i, 0)),
            pl.BlockSpec((_BR, 2 * _NC), lambda i: (i, 0)),
            pl.BlockSpec((_H, _H), lambda i: (0, 0)),
        ],
        out_specs=pl.BlockSpec((_BR, _H), lambda i: (i, 0)),
        out_shape=jax.ShapeDtypeStruct((_N, _H), jnp.float32),
    )(p, x0, degp, Wc)


def _tc_final_body(beta, p_ref, x0_ref, deg_ref, wc_ref, w1_ref, b1_ref,
                   out_ref):
    _, nd = _norms(deg_ref[...])
    p = p_ref[...]
    agg = (p[0] + p[1]) * nd[:, None]
    rst = (1.0 - _ALPHA) * agg + _ALPHA * x0_ref[...]
    h = (1.0 - beta) * rst + beta * jnp.dot(
        rst, wc_ref[...], preferred_element_type=jnp.float32)
    out_ref[...] = jnp.dot(
        h, w1_ref[...], preferred_element_type=jnp.float32) + b1_ref[...]


def _tc_final(beta, p, x0, degp, Wc, W1, b1r):
    return pl.pallas_call(
        functools.partial(_tc_final_body, beta),
        grid=(_GRID,),
        in_specs=[
            pl.BlockSpec((_NC, _BR, _H), lambda i: (0, i, 0)),
            pl.BlockSpec((_BR, _H), lambda i: (i, 0)),
            pl.BlockSpec((_BR, 2 * _NC), lambda i: (i, 0)),
            pl.BlockSpec((_H, _H), lambda i: (0, 0)),
            pl.BlockSpec((_H, _C), lambda i: (0, 0)),
            pl.BlockSpec((1, _C), lambda i: (0, 0)),
        ],
        out_specs=pl.BlockSpec((_BR, _C), lambda i: (i, 0)),
        out_shape=jax.ShapeDtypeStruct((_N, _C), jnp.float32),
    )(p, x0, degp, Wc, W1, b1r)


def kernel(x, edge_index, W0, b0, Wc1, Wc2, Wc3, W1, b1):
    src = edge_index[0]
    dst = edge_index[1]
    pad = _EPAD - _E

    pad_n = jnp.full((pad,), _N, jnp.int32)
    srcs_deg = jnp.concatenate([src, pad_n]).reshape(_NW, _CPT, _CH)
    dsts_deg = jnp.concatenate([dst, pad_n]).reshape(_NW, _CPT, _CH)

    pad_a = _EPA - _E
    epk = jnp.concatenate(
        [src | (dst << _PK), jnp.full((pad_a,), _N << _PK, jnp.int32)]
    ).reshape(_NW, _CPA, _CHA)

    degp = _deg_kernel(srcs_deg, dsts_deg)
    degp = degp[:, :, :_N].transpose(2, 0, 1).reshape(_N, 2 * _NC)
    x0, m = _tc_in(x, W0, b0.reshape(1, _H), degp)
    for l, Wc in enumerate((Wc1, Wc2, Wc3), start=1):
        beta = math.log(_LAMBDA / l + 1.0)
        p = _agg_kernel(m, epk)
        if l < 3:
            m = _tc_layer(beta, p, x0, degp, Wc)
        else:
            out = _tc_final(beta, p, x0, degp, Wc, W1, b1.reshape(1, _C))
    return out


# speedup vs baseline: 1.8254x; 1.8254x over previous
import functools
import math

import jax
import jax.numpy as jnp
from jax import lax
from jax.experimental import pallas as pl
from jax.experimental.pallas import tpu as pltpu
from jax.experimental.pallas import tpu_sc as plsc

_N = 10000
_E = 320000
_D = 128
_H = 128
_C = 64
_ALPHA = 0.1
_LAMBDA = 0.5

_NC = 2
_NS = 16
_NW = _NC * _NS
_CHUNK = 128
_CPT = 80
_EPAD = _NW * _CPT * _CHUNK
_NPAD = 10240
_ZR = _NPAD // _NS


def _sc_mesh():
    return plsc.VectorSubcoreMesh(
        core_axis_name="c", subcore_axis_name="s",
        num_cores=_NC, num_subcores=_NS)


@functools.partial(
    pl.kernel,
    out_type=jax.ShapeDtypeStruct((_NC, 2, _NPAD), jnp.float32),
    mesh=_sc_mesh(),
    scratch_types=[
        pltpu.VMEM((_CPT, _CHUNK), jnp.int32),
        pltpu.VMEM((_CPT, _CHUNK), jnp.int32),
        pltpu.VMEM((_CHUNK,), jnp.float32),
        pltpu.VMEM((_ZR,), jnp.float32),
        pltpu.VMEM_SHARED((_NPAD,), jnp.float32),
        pltpu.VMEM_SHARED((_NPAD,), jnp.float32),
    ],
)
def _deg_kernel(srcs, dsts, out, src_v, dst_v, ones_v, zer_v, do_sh, di_sh):
    cid = lax.axis_index("c")
    sid = lax.axis_index("s")
    wid = cid * _NS + sid
    for k in range(_CHUNK // 16):
        ones_v[pl.ds(k * 16, 16)] = jnp.full((16,), 1.0, jnp.float32)
    for k in range(_ZR // 16):
        zer_v[pl.ds(k * 16, 16)] = jnp.zeros((16,), jnp.float32)
    pltpu.sync_copy(zer_v, do_sh.at[pl.ds(sid * _ZR, _ZR)])
    pltpu.sync_copy(zer_v, di_sh.at[pl.ds(sid * _ZR, _ZR)])
    pltpu.sync_copy(srcs.at[wid], src_v)
    pltpu.sync_copy(dsts.at[wid], dst_v)
    plsc.subcore_barrier()

    def body(j, carry):
        pltpu.sync_copy(ones_v, do_sh.at[src_v.at[j]], add=True)
        pltpu.sync_copy(ones_v, di_sh.at[dst_v.at[j]], add=True)
        return carry

    lax.fori_loop(0, _CPT, body, 0)
    plsc.subcore_barrier()
    rows = pl.ds(sid * _ZR, _ZR)
    pltpu.sync_copy(do_sh.at[rows], out.at[cid, 0, rows])
    pltpu.sync_copy(di_sh.at[rows], out.at[cid, 1, rows])


_HH = _H // _NC
_CPS = _EPAD // _CHUNK // _NS


@functools.partial(
    pl.kernel,
    out_type=jax.ShapeDtypeStruct((_NC, _NPAD, _HH), jnp.float32),
    mesh=_sc_mesh(),
    scratch_types=[
        pltpu.VMEM((_CPS, _CHUNK), jnp.int32),
        pltpu.VMEM((_CPS, _CHUNK), jnp.int32),
        pltpu.VMEM((_CHUNK, _HH), jnp.float32),
        pltpu.VMEM((_CHUNK, _HH), jnp.float32),
        pltpu.VMEM((_CHUNK, _HH), jnp.float32),
        pltpu.VMEM((_CHUNK, _HH), jnp.float32),
        pltpu.VMEM_SHARED((_NPAD, _HH), jnp.float32),
        pltpu.SemaphoreType.DMA,
        pltpu.SemaphoreType.DMA,
        pltpu.SemaphoreType.DMA,
        pltpu.SemaphoreType.DMA,
        pltpu.SemaphoreType.DMA,
        pltpu.SemaphoreType.DMA,
        pltpu.SemaphoreType.DMA,
        pltpu.SemaphoreType.DMA,
    ],
    compiler_params=pltpu.CompilerParams(use_tc_tiling_on_sc=False),
)
def _agg_kernel(m_hbm, srcs, dsts, zrows, out,
                src_v, dst_v, rows0, rows1, rows2, rows3, acc_sh,
                sg0, sg1, sg2, sg3, ss0, ss1, ss2, ss3):
    rows = (rows0, rows1, rows2, rows3)
    sg = (sg0, sg1, sg2, sg3)
    ss = (ss0, ss1, ss2, ss3)
    cid = lax.axis_index("c")
    sid = lax.axis_index("s")
    mh = m_hbm.at[cid]
    pltpu.sync_copy(zrows, acc_sh.at[pl.ds(sid * _ZR, _ZR)])
    pltpu.sync_copy(srcs.at[sid], src_v)
    pltpu.sync_copy(dsts.at[sid], dst_v)
    plsc.subcore_barrier()

    def _wait_gather(j, b):
        pltpu.make_async_copy(mh.at[src_v.at[j]], rows[b], sg[b]).wait()

    def _wait_scatter(j, b):
        pltpu.make_async_copy(rows[b], acc_sh.at[dst_v.at[j]], ss[b]).wait()

    def _start_scatter(j, b):
        pltpu.async_copy(rows[b], acc_sh.at[dst_v.at[j]], ss[b], add=True)

    def body(g, carry):
        j0 = g * 4
        for b in range(4):
            j = j0 + b

            @pl.when(g > 0)
            def _(b=b, j=j):
                _wait_scatter(j - 4, b)

            pltpu.async_copy(mh.at[src_v.at[j]], rows[b], sg[b])
            bs = (b + 2) % 4
            js = j - 2
            if b < 2:
                @pl.when(g > 0)
                def _(bs=bs, js=js):
                    _wait_gather(js, bs)
                    _start_scatter(js, bs)
            else:
                _wait_gather(js, bs)
                _start_scatter(js, bs)
        return carry

    lax.fori_loop(0, _CPS // 4, body, 0)
    for jj, b in ((_CPS - 2, 2), (_CPS - 1, 3)):
        _wait_gather(jj, b)
        _start_scatter(jj, b)
    for b in range(4):
        _wait_scatter(_CPS - 4 + b, b)
    plsc.subcore_barrier()
    r = pl.ds(sid * _ZR, _ZR)
    pltpu.sync_copy(acc_sh.at[r], out.at[cid, r])


_BR = 1000
_GRID = _N // _BR


def _norms(deg_blk):
    deg_out = deg_blk[:, 0] + deg_blk[:, 2]
    deg_in = deg_blk[:, 1] + deg_blk[:, 3]
    ns = lax.rsqrt(jnp.maximum(deg_out, 1.0))
    nd = lax.rsqrt(jnp.maximum(deg_in, 1.0))
    return ns, nd


def _split_m(m_ref, m2):
    m_ref[0] = m2[:, :_HH]
    m_ref[1] = m2[:, _HH:]


def _tc_in_body(x_ref, w_ref, b_ref, deg_ref, x0_ref, m_ref):
    h = jnp.dot(x_ref[...], w_ref[...], preferred_element_type=jnp.float32)
    h = jnp.maximum(h + b_ref[...], 0.0)
    ns, _ = _norms(deg_ref[...])
    x0_ref[...] = h
    _split_m(m_ref, h * ns[:, None])


def _tc_in(x, W0, b0r, degp):
    return pl.pallas_call(
        _tc_in_body,
        grid=(_GRID,),
        in_specs=[
            pl.BlockSpec((_BR, _D), lambda i: (i, 0)),
            pl.BlockSpec((_D, _H), lambda i: (0, 0)),
            pl.BlockSpec((1, _H), lambda i: (0, 0)),
            pl.BlockSpec((_BR, 2 * _NC), lambda i: (i, 0)),
        ],
        out_specs=[
            pl.BlockSpec((_BR, _H), lambda i: (i, 0)),
            pl.BlockSpec((_NC, _BR, _HH), lambda i: (0, i, 0)),
        ],
        out_shape=[
            jax.ShapeDtypeStruct((_N, _H), jnp.float32),
            jax.ShapeDtypeStruct((_NC, _N, _HH), jnp.float32),
        ],
    )(x, W0, b0r, degp)


def _agg_from_p(p_ref, deg_blk):
    ns, nd = _norms(deg_blk)
    p = p_ref[...]
    agg = jnp.concatenate([p[0], p[1]], axis=1) * nd[:, None]
    return agg, ns


def _tc_layer_body(beta, p_ref, x0_ref, deg_ref, wc_ref, m_ref):
    agg, ns = _agg_from_p(p_ref, deg_ref[...])
    rst = (1.0 - _ALPHA) * agg + _ALPHA * x0_ref[...]
    h = (1.0 - beta) * rst + beta * jnp.dot(
        rst, wc_ref[...], preferred_element_type=jnp.float32)
    h = jnp.maximum(h, 0.0)
    _split_m(m_ref, h * ns[:, None])


def _tc_layer(beta, p, x0, degp, Wc):
    return pl.pallas_call(
        functools.partial(_tc_layer_body, beta),
        grid=(_GRID,),
        in_specs=[
            pl.BlockSpec((_NC, _BR, _HH), lambda i: (0, i, 0)),
            pl.BlockSpec((_BR, _H), lambda i: (i, 0)),
            pl.BlockSpec((_BR, 2 * _NC), lambda i: (i, 0)),
            pl.BlockSpec((_H, _H), lambda i: (0, 0)),
        ],
        out_specs=pl.BlockSpec((_NC, _BR, _HH), lambda i: (0, i, 0)),
        out_shape=jax.ShapeDtypeStruct((_NC, _N, _HH), jnp.float32),
    )(p, x0, degp, Wc)


def _tc_final_body(beta, p_ref, x0_ref, deg_ref, wc_ref, w1_ref, b1_ref,
                   out_ref):
    agg, _ = _agg_from_p(p_ref, deg_ref[...])
    rst = (1.0 - _ALPHA) * agg + _ALPHA * x0_ref[...]
    h = (1.0 - beta) * rst + beta * jnp.dot(
        rst, wc_ref[...], preferred_element_type=jnp.float32)
    out_ref[...] = jnp.dot(
        h, w1_ref[...], preferred_element_type=jnp.float32) + b1_ref[...]


def _tc_final(beta, p, x0, degp, Wc, W1, b1r):
    return pl.pallas_call(
        functools.partial(_tc_final_body, beta),
        grid=(_GRID,),
        in_specs=[
            pl.BlockSpec((_NC, _BR, _HH), lambda i: (0, i, 0)),
            pl.BlockSpec((_BR, _H), lambda i: (i, 0)),
            pl.BlockSpec((_BR, 2 * _NC), lambda i: (i, 0)),
            pl.BlockSpec((_H, _H), lambda i: (0, 0)),
            pl.BlockSpec((_H, _C), lambda i: (0, 0)),
            pl.BlockSpec((1, _C), lambda i: (0, 0)),
        ],
        out_specs=pl.BlockSpec((_BR, _C), lambda i: (i, 0)),
        out_shape=jax.ShapeDtypeStruct((_N, _C), jnp.float32),
    )(p, x0, degp, Wc, W1, b1r)


def kernel(x, edge_index, W0, b0, Wc1, Wc2, Wc3, W1, b1):
    src = edge_index[0]
    dst = edge_index[1]
    pad = _EPAD - _E
    pad_n = jnp.full((pad,), _N, jnp.int32)
    srcs_deg = jnp.concatenate([src, pad_n]).reshape(_NW, _CPT, _CHUNK)
    dst_p = jnp.concatenate([dst, pad_n])
    dsts_deg = dst_p.reshape(_NW, _CPT, _CHUNK)
    dsts_agg = dst_p.reshape(_NS, _CPS, _CHUNK)
    srcs_agg = jnp.concatenate(
        [src, jnp.zeros((pad,), jnp.int32)]).reshape(_NS, _CPS, _CHUNK)
    zrows = jnp.zeros((_ZR, _HH), jnp.float32)

    degp = _deg_kernel(srcs_deg, dsts_deg)
    degp = degp[:, :, :_N].transpose(2, 0, 1).reshape(_N, 2 * _NC)
    x0, m = _tc_in(x, W0, b0.reshape(1, _H), degp)
    for l, Wc in enumerate((Wc1, Wc2, Wc3), start=1):
        beta = math.log(_LAMBDA / l + 1.0)
        p = _agg_kernel(m, srcs_agg, dsts_agg, zrows)
        if l < 3:
            m = _tc_layer(beta, p, x0, degp, Wc)
        else:
            out = _tc_final(beta, p, x0, degp, Wc, W1, b1.reshape(1, _C))
    return out


# speedup vs baseline: 1.8410x; 1.0086x over previous
import functools
import math

import jax
import jax.numpy as jnp
from jax import lax
from jax.experimental import pallas as pl
from jax.experimental.pallas import tpu as pltpu
from jax.experimental.pallas import tpu_sc as plsc

_N = 10000
_E = 320000
_D = 128
_H = 128
_C = 64
_ALPHA = 0.1
_LAMBDA = 0.5

_NC = 2
_NS = 16
_NW = _NC * _NS
_CHUNK = 128
_CPT = 80
_EPAD = _NW * _CPT * _CHUNK
_NPAD = 10240
_ZR = _NPAD // _NS


def _sc_mesh():
    return plsc.VectorSubcoreMesh(
        core_axis_name="c", subcore_axis_name="s",
        num_cores=_NC, num_subcores=_NS)


@functools.partial(
    pl.kernel,
    out_type=jax.ShapeDtypeStruct((_NC, 2, _NPAD), jnp.float32),
    mesh=_sc_mesh(),
    scratch_types=[
        pltpu.VMEM((_CPT, _CHUNK), jnp.int32),
        pltpu.VMEM((_CPT, _CHUNK), jnp.int32),
        pltpu.VMEM((_CHUNK,), jnp.float32),
        pltpu.VMEM((_ZR,), jnp.float32),
        pltpu.VMEM_SHARED((_NPAD,), jnp.float32),
        pltpu.VMEM_SHARED((_NPAD,), jnp.float32),
    ],
)
def _deg_kernel(srcs, dsts, out, src_v, dst_v, ones_v, zer_v, do_sh, di_sh):
    cid = lax.axis_index("c")
    sid = lax.axis_index("s")
    wid = cid * _NS + sid
    for k in range(_CHUNK // 16):
        ones_v[pl.ds(k * 16, 16)] = jnp.full((16,), 1.0, jnp.float32)
    for k in range(_ZR // 16):
        zer_v[pl.ds(k * 16, 16)] = jnp.zeros((16,), jnp.float32)
    pltpu.sync_copy(zer_v, do_sh.at[pl.ds(sid * _ZR, _ZR)])
    pltpu.sync_copy(zer_v, di_sh.at[pl.ds(sid * _ZR, _ZR)])
    pltpu.sync_copy(srcs.at[wid], src_v)
    pltpu.sync_copy(dsts.at[wid], dst_v)
    plsc.subcore_barrier()

    def body(j, carry):
        pltpu.sync_copy(ones_v, do_sh.at[src_v.at[j]], add=True)
        pltpu.sync_copy(ones_v, di_sh.at[dst_v.at[j]], add=True)
        return carry

    lax.fori_loop(0, _CPT, body, 0)
    plsc.subcore_barrier()
    rows = pl.ds(sid * _ZR, _ZR)
    pltpu.sync_copy(do_sh.at[rows], out.at[cid, 0, rows])
    pltpu.sync_copy(di_sh.at[rows], out.at[cid, 1, rows])


_HH = _H // _NC
_CPS = _EPAD // _CHUNK // _NS


@functools.partial(
    pl.kernel,
    out_type=jax.ShapeDtypeStruct((_NC, _NPAD, _HH), jnp.float32),
    mesh=_sc_mesh(),
    scratch_types=[
        pltpu.VMEM((_CPS, _CHUNK), jnp.int32),
        pltpu.VMEM((_CPS, _CHUNK), jnp.int32),
        pltpu.VMEM((_CHUNK, _HH), jnp.float32),
        pltpu.VMEM((_CHUNK, _HH), jnp.float32),
        pltpu.VMEM((_CHUNK, _HH), jnp.float32),
        pltpu.VMEM((_CHUNK, _HH), jnp.float32),
        pltpu.VMEM((_CHUNK, _HH), jnp.float32),
        pltpu.VMEM_SHARED((_NPAD, _HH), jnp.float32),
        pltpu.SemaphoreType.DMA,
        pltpu.SemaphoreType.DMA,
        pltpu.SemaphoreType.DMA,
        pltpu.SemaphoreType.DMA,
        pltpu.SemaphoreType.DMA,
        pltpu.SemaphoreType.DMA,
        pltpu.SemaphoreType.DMA,
        pltpu.SemaphoreType.DMA,
        pltpu.SemaphoreType.DMA,
        pltpu.SemaphoreType.DMA,
    ],
    compiler_params=pltpu.CompilerParams(use_tc_tiling_on_sc=False),
)
def _agg_kernel(m_hbm, srcs, dsts, zrows, out,
                src_v, dst_v, rows0, rows1, rows2, rows3, rows4, acc_sh,
                sg0, sg1, sg2, sg3, sg4, ss0, ss1, ss2, ss3, ss4):
    rows = (rows0, rows1, rows2, rows3, rows4)
    sg = (sg0, sg1, sg2, sg3, sg4)
    ss = (ss0, ss1, ss2, ss3, ss4)
    cid = lax.axis_index("c")
    sid = lax.axis_index("s")
    mh = m_hbm.at[cid]
    pltpu.sync_copy(zrows, acc_sh.at[pl.ds(sid * _ZR, _ZR)])
    pltpu.sync_copy(srcs.at[sid], src_v)
    pltpu.sync_copy(dsts.at[sid], dst_v)
    plsc.subcore_barrier()

    def _wait_gather(j, b):
        pltpu.make_async_copy(mh.at[src_v.at[j]], rows[b], sg[b]).wait()

    def _wait_scatter(j, b):
        pltpu.make_async_copy(rows[b], acc_sh.at[dst_v.at[j]], ss[b]).wait()

    def _start_scatter(j, b):
        pltpu.async_copy(rows[b], acc_sh.at[dst_v.at[j]], ss[b], add=True)

    def body(g, carry):
        j0 = g * 5
        for b in range(5):
            j = j0 + b

            @pl.when(g > 0)
            def _(b=b, j=j):
                _wait_scatter(j - 5, b)

            pltpu.async_copy(mh.at[src_v.at[j]], rows[b], sg[b])
            bs = (b + 2) % 5
            js = j - 3
            if b < 3:
                @pl.when(g > 0)
                def _(bs=bs, js=js):
                    _wait_gather(js, bs)
                    _start_scatter(js, bs)
            else:
                _wait_gather(js, bs)
                _start_scatter(js, bs)
        return carry

    lax.fori_loop(0, _CPS // 5, body, 0)
    for jj in (_CPS - 3, _CPS - 2, _CPS - 1):
        _wait_gather(jj, jj % 5)
        _start_scatter(jj, jj % 5)
    for b in range(5):
        _wait_scatter(_CPS - 5 + b, b)
    plsc.subcore_barrier()
    r = pl.ds(sid * _ZR, _ZR)
    pltpu.sync_copy(acc_sh.at[r], out.at[cid, r])


_BR = 1000
_GRID = _N // _BR


def _norms(deg_blk):
    deg_out = deg_blk[:, 0] + deg_blk[:, 2]
    deg_in = deg_blk[:, 1] + deg_blk[:, 3]
    ns = lax.rsqrt(jnp.maximum(deg_out, 1.0))
    nd = lax.rsqrt(jnp.maximum(deg_in, 1.0))
    return ns, nd


def _split_m(m_ref, m2):
    m_ref[0] = m2[:, :_HH]
    m_ref[1] = m2[:, _HH:]


def _tc_in_body(x_ref, w_ref, b_ref, deg_ref, x0_ref, m_ref):
    h = jnp.dot(x_ref[...], w_ref[...], preferred_element_type=jnp.float32)
    h = jnp.maximum(h + b_ref[...], 0.0)
    ns, _ = _norms(deg_ref[...])
    x0_ref[...] = h
    _split_m(m_ref, h * ns[:, None])


def _tc_in(x, W0, b0r, degp):
    return pl.pallas_call(
        _tc_in_body,
        grid=(_GRID,),
        in_specs=[
            pl.BlockSpec((_BR, _D), lambda i: (i, 0)),
            pl.BlockSpec((_D, _H), lambda i: (0, 0)),
            pl.BlockSpec((1, _H), lambda i: (0, 0)),
            pl.BlockSpec((_BR, 2 * _NC), lambda i: (i, 0)),
        ],
        out_specs=[
            pl.BlockSpec((_BR, _H), lambda i: (i, 0)),
            pl.BlockSpec((_NC, _BR, _HH), lambda i: (0, i, 0)),
        ],
        out_shape=[
            jax.ShapeDtypeStruct((_N, _H), jnp.float32),
            jax.ShapeDtypeStruct((_NC, _N, _HH), jnp.float32),
        ],
    )(x, W0, b0r, degp)


def _agg_from_p(p_ref, deg_blk):
    ns, nd = _norms(deg_blk)
    p = p_ref[...]
    agg = jnp.concatenate([p[0], p[1]], axis=1) * nd[:, None]
    return agg, ns


def _tc_layer_body(beta, p_ref, x0_ref, deg_ref, wc_ref, m_ref):
    agg, ns = _agg_from_p(p_ref, deg_ref[...])
    rst = (1.0 - _ALPHA) * agg + _ALPHA * x0_ref[...]
    h = (1.0 - beta) * rst + beta * jnp.dot(
        rst, wc_ref[...], preferred_element_type=jnp.float32)
    h = jnp.maximum(h, 0.0)
    _split_m(m_ref, h * ns[:, None])


def _tc_layer(beta, p, x0, degp, Wc):
    return pl.pallas_call(
        functools.partial(_tc_layer_body, beta),
        grid=(_GRID,),
        in_specs=[
            pl.BlockSpec((_NC, _BR, _HH), lambda i: (0, i, 0)),
            pl.BlockSpec((_BR, _H), lambda i: (i, 0)),
            pl.BlockSpec((_BR, 2 * _NC), lambda i: (i, 0)),
            pl.BlockSpec((_H, _H), lambda i: (0, 0)),
        ],
        out_specs=pl.BlockSpec((_NC, _BR, _HH), lambda i: (0, i, 0)),
        out_shape=jax.ShapeDtypeStruct((_NC, _N, _HH), jnp.float32),
    )(p, x0, degp, Wc)


def _tc_final_body(beta, p_ref, x0_ref, deg_ref, wc_ref, w1_ref, b1_ref,
                   out_ref):
    agg, _ = _agg_from_p(p_ref, deg_ref[...])
    rst = (1.0 - _ALPHA) * agg + _ALPHA * x0_ref[...]
    h = (1.0 - beta) * rst + beta * jnp.dot(
        rst, wc_ref[...], preferred_element_type=jnp.float32)
    out_ref[...] = jnp.dot(
        h, w1_ref[...], preferred_element_type=jnp.float32) + b1_ref[...]


def _tc_final(beta, p, x0, degp, Wc, W1, b1r):
    return pl.pallas_call(
        functools.partial(_tc_final_body, beta),
        grid=(_GRID,),
        in_specs=[
            pl.BlockSpec((_NC, _BR, _HH), lambda i: (0, i, 0)),
            pl.BlockSpec((_BR, _H), lambda i: (i, 0)),
            pl.BlockSpec((_BR, 2 * _NC), lambda i: (i, 0)),
            pl.BlockSpec((_H, _H), lambda i: (0, 0)),
            pl.BlockSpec((_H, _C), lambda i: (0, 0)),
            pl.BlockSpec((1, _C), lambda i: (0, 0)),
        ],
        out_specs=pl.BlockSpec((_BR, _C), lambda i: (i, 0)),
        out_shape=jax.ShapeDtypeStruct((_N, _C), jnp.float32),
    )(p, x0, degp, Wc, W1, b1r)


def kernel(x, edge_index, W0, b0, Wc1, Wc2, Wc3, W1, b1):
    src = edge_index[0]
    dst = edge_index[1]
    pad = _EPAD - _E
    pad_n = jnp.full((pad,), _N, jnp.int32)
    srcs_deg = jnp.concatenate([src, pad_n]).reshape(_NW, _CPT, _CHUNK)
    dst_p = jnp.concatenate([dst, pad_n])
    dsts_deg = dst_p.reshape(_NW, _CPT, _CHUNK)
    dsts_agg = dst_p.reshape(_NS, _CPS, _CHUNK)
    srcs_agg = jnp.concatenate(
        [src, jnp.zeros((pad,), jnp.int32)]).reshape(_NS, _CPS, _CHUNK)
    zrows = jnp.zeros((_ZR, _HH), jnp.float32)

    degp = _deg_kernel(srcs_deg, dsts_deg)
    degp = degp[:, :, :_N].transpose(2, 0, 1).reshape(_N, 2 * _NC)
    x0, m = _tc_in(x, W0, b0.reshape(1, _H), degp)
    for l, Wc in enumerate((Wc1, Wc2, Wc3), start=1):
        beta = math.log(_LAMBDA / l + 1.0)
        p = _agg_kernel(m, srcs_agg, dsts_agg, zrows)
        if l < 3:
            m = _tc_layer(beta, p, x0, degp, Wc)
        else:
            out = _tc_final(beta, p, x0, degp, Wc, W1, b1.reshape(1, _C))
    return out
